# baseline (speedup 1.0000x reference)
import functools

import jax
import jax.numpy as jnp
from jax import lax
from jax.experimental import pallas as pl
from jax.experimental.pallas import tpu as pltpu
from jax.experimental.pallas import tpu_sc as plsc

N_SUP = 10000
N_PROD = 50000
N_WH = 10000
H = 128

NC = 2
NS = 16
NW = NC * NS

CH = 128

RP = 50048
RPT = RP // NS
RW = 10240
RWT = RW // NS

E1P = 327680
E2P = 294912


def _zero_rows(zbuf, zflat, acc, cnta, base, nrows, zr, with_cnt):
    full, tail = nrows // zr, nrows % zr
    for z in range(full):
        pltpu.sync_copy(zbuf, acc.at[pl.ds(base + z * zr, zr)])
        if with_cnt:
            pltpu.sync_copy(zflat, cnta.at[pl.ds(base + z * zr, zr)])
    if tail:
        pltpu.sync_copy(zbuf.at[pl.ds(0, tail)],
                        acc.at[pl.ds(base + full * zr, tail)])
        if with_cnt:
            pltpu.sync_copy(zflat.at[pl.ds(0, tail)],
                            cnta.at[pl.ds(base + full * zr, tail)])


def _proj_body(x_ref, w_ref, b_ref, o_ref):
    o_ref[...] = jnp.maximum(
        jnp.dot(x_ref[...], w_ref[...], preferred_element_type=jnp.float32)
        + b_ref[...], 0.0)


def _proj(x, W, b, bm=512):
    n, d = x.shape
    h = W.shape[1]
    return pl.pallas_call(
        _proj_body,
        grid=(pl.cdiv(n, bm),),
        in_specs=[
            pl.BlockSpec((bm, d), lambda i: (i, 0)),
            pl.BlockSpec((d, h), lambda i: (0, 0)),
            pl.BlockSpec((1, h), lambda i: (0, 0)),
        ],
        out_specs=pl.BlockSpec((bm, h), lambda i: (i, 0)),
        out_shape=jax.ShapeDtypeStruct((n, h), jnp.float32),
    )(x, W, b.reshape(1, h))


def _combine_body(p0_ref, p1_ref, c0_ref, c1_ref, x_ref, wl_ref, wr_ref,
                  b_ref, o_ref):
    cnt = jnp.maximum(c0_ref[...] + c1_ref[...], 1.0)
    m = (p0_ref[...] + p1_ref[...]) / cnt
    o_ref[...] = jnp.maximum(
        jnp.dot(m, wl_ref[...], preferred_element_type=jnp.float32)
        + jnp.dot(x_ref[...], wr_ref[...], preferred_element_type=jnp.float32)
        + b_ref[...], 0.0)


def _combine1_body(p_ref, c_ref, x_ref, wl_ref, wr_ref, b_ref, o_ref):
    cnt = jnp.maximum(c_ref[...], 1.0)
    m = p_ref[...] / cnt
    o_ref[...] = jnp.maximum(
        jnp.dot(m, wl_ref[...], preferred_element_type=jnp.float32)
        + jnp.dot(x_ref[...], wr_ref[...], preferred_element_type=jnp.float32)
        + b_ref[...], 0.0)


def _combine1(p, cnt, x, Wl, Wr, b, bm=512):
    n = x.shape[0]
    return pl.pallas_call(
        _combine1_body,
        grid=(pl.cdiv(n, bm),),
        in_specs=[
            pl.BlockSpec((bm, H), lambda i: (i, 0)),
            pl.BlockSpec((bm, 1), lambda i: (i, 0)),
            pl.BlockSpec((bm, H), lambda i: (i, 0)),
            pl.BlockSpec((H, H), lambda i: (0, 0)),
            pl.BlockSpec((H, H), lambda i: (0, 0)),
            pl.BlockSpec((1, H), lambda i: (0, 0)),
        ],
        out_specs=pl.BlockSpec((bm, H), lambda i: (i, 0)),
        out_shape=jax.ShapeDtypeStruct((n, H), jnp.float32),
    )(p, cnt, x, Wl, Wr, b.reshape(1, H))


def _combine(p0, p1, c0, c1, x, Wl, Wr, b, bm=512):
    n = x.shape[0]
    return pl.pallas_call(
        _combine_body,
        grid=(pl.cdiv(n, bm),),
        in_specs=[
            pl.BlockSpec((bm, H), lambda i: (i, 0)),
            pl.BlockSpec((bm, H), lambda i: (i, 0)),
            pl.BlockSpec((bm, 1), lambda i: (i, 0)),
            pl.BlockSpec((bm, 1), lambda i: (i, 0)),
            pl.BlockSpec((bm, H), lambda i: (i, 0)),
            pl.BlockSpec((H, H), lambda i: (0, 0)),
            pl.BlockSpec((H, H), lambda i: (0, 0)),
            pl.BlockSpec((1, H), lambda i: (0, 0)),
        ],
        out_specs=pl.BlockSpec((bm, H), lambda i: (i, 0)),
        out_shape=jax.ShapeDtypeStruct((n, H), jnp.float32),
    )(p0, p1, c0, c1, x, Wl, Wr, b.reshape(1, H))


def _seg_pipeline(edges, row0, nch, table, acc, cnta, ibuf, rbuf, onev,
                  isem, gsem, ssem, with_cnt):
    def idx_load(j, sl):
        pltpu.async_copy(edges.at[row0 + j], ibuf.at[sl], isem.at[sl])

    def idx_wait(sl):
        pltpu.make_async_copy(edges.at[0], ibuf.at[sl], isem.at[sl]).wait()

    def gather(sl, rs):
        pltpu.async_copy(table.at[ibuf.at[sl, 0]], rbuf.at[rs], gsem.at[rs])

    def gather_wait(sl, rs):
        pltpu.make_async_copy(table.at[ibuf.at[sl, 0]], rbuf.at[rs],
                              gsem.at[rs]).wait()

    def scat(sl, rs):
        pltpu.async_copy(rbuf.at[rs], acc.at[ibuf.at[sl, 1]], ssem.at[rs],
                         add=True)
        if with_cnt:
            pltpu.async_copy(onev, cnta.at[ibuf.at[sl, 1]], ssem.at[rs],
                             add=True)

    def scat_wait(sl, rs):
        pltpu.make_async_copy(rbuf.at[rs], acc.at[ibuf.at[sl, 1]],
                              ssem.at[rs]).wait()
        if with_cnt:
            pltpu.make_async_copy(onev, cnta.at[ibuf.at[sl, 1]],
                                  ssem.at[rs]).wait()

    def step(j, u, prefetch):
        rs = u % 2
        scat_wait((u - 2) % 4, rs)
        if prefetch:
            idx_load(j + 2, (u + 2) % 4)
        idx_wait(u)
        gather(u, rs)
        gather_wait((u - 1) % 4, 1 - rs)
        scat((u - 1) % 4, 1 - rs)

    for sl in range(4):
        idx_load(sl, sl)
    idx_wait(0)
    gather(0, 0)
    idx_wait(1)
    gather(1, 1)
    gather_wait(0, 0)
    scat(0, 0)

    def grp(m, carry):
        j0 = 2 + 4 * m
        step(j0, 2, True)
        step(j0 + 1, 3, True)
        step(j0 + 2, 0, True)
        step(j0 + 3, 1, True)
        return carry

    lax.fori_loop(0, (nch - 4) // 4, grp, 0)
    step(nch - 2, 2, False)
    step(nch - 1, 3, False)
    gather_wait(3, 1)
    scat(3, 1)
    scat_wait(2, 0)
    scat_wait(3, 1)


def _seg_pipeline4(edges, row0, nch, table, acc, cnta, ibuf, rbuf, onev,
                   isem, gsem, ssem, with_cnt):
    def idx_load(j, sl):
        pltpu.async_copy(edges.at[row0 + j], ibuf.at[sl], isem.at[sl])

    def idx_wait(sl):
        pltpu.make_async_copy(edges.at[0], ibuf.at[sl], isem.at[sl]).wait()

    def gather(sl):
        pltpu.async_copy(table.at[ibuf.at[sl, 0]], rbuf.at[sl], gsem.at[sl])

    def gather_wait(sl):
        pltpu.make_async_copy(table.at[ibuf.at[sl, 0]], rbuf.at[sl],
                              gsem.at[sl]).wait()

    def scat(sl):
        pltpu.async_copy(rbuf.at[sl], acc.at[ibuf.at[sl, 1]], ssem.at[sl],
                         add=True)
        if with_cnt:
            pltpu.async_copy(onev, cnta.at[ibuf.at[sl, 1]], ssem.at[sl],
                             add=True)

    def scat_wait(sl):
        pltpu.make_async_copy(rbuf.at[sl], acc.at[ibuf.at[sl, 1]],
                              ssem.at[sl]).wait()
        if with_cnt:
            pltpu.make_async_copy(onev, cnta.at[ibuf.at[sl, 1]],
                                  ssem.at[sl]).wait()

    def step(j, u, first=False, prefetch=True, gather_next=True):
        if not first:
            scat_wait((u + 2) % 4)
        if prefetch:
            idx_load(j + 2, (u + 2) % 4)
        if gather_next:
            idx_wait((u + 1) % 4)
            gather((u + 1) % 4)
        gather_wait(u)
        scat(u)

    idx_load(0, 0)
    idx_load(1, 1)
    idx_wait(0)
    gather(0)
    step(0, 0, first=True)
    step(1, 1, first=True)

    def grp(m, carry):
        j0 = 2 + 4 * m
        step(j0, 2)
        step(j0 + 1, 3)
        step(j0 + 2, 0)
        step(j0 + 3, 1)
        return carry

    lax.fori_loop(0, (nch - 4) // 4, grp, 0)
    step(nch - 2, 2, prefetch=False)
    step(nch - 1, 3, prefetch=False, gather_next=False)
    scat_wait(2)
    scat_wait(3)


def _agg_sup_body(t0, t1, t2, t3, edges, zrows, z1, ones,
                  psum, pcnt,
                  acc, cnta, ibuf, rbuf, onev, zflat, isem, gsem, ssem):
    c = lax.axis_index("c")
    s = lax.axis_index("s")
    nch = E1P // (NS * CH)
    row0 = s * nch

    pltpu.sync_copy(ones, onev)
    pltpu.sync_copy(z1, zflat)

    for p in range(2):
        pltpu.sync_copy(zrows, rbuf.at[0])
        _zero_rows(rbuf.at[0], zflat, acc, cnta, s * RPT, RPT, CH, p == 0)
        plsc.subcore_barrier()
        for cc in range(NC):
            tf = (t0, t1, t2, t3)[2 * cc + p]
            wc = p == 0 and cc == 0

            @pl.when(c == cc)
            def _():
                _seg_pipeline4(edges, row0, nch, tf, acc, cnta, ibuf, rbuf,
                               onev, isem, gsem, ssem, wc)

        plsc.subcore_barrier()
        for cc in range(NC):
            col = (2 * cc + p) * 32

            @pl.when(c == cc)
            def _():
                pltpu.sync_copy(
                    acc.at[pl.ds(s * RPT, RPT)],
                    psum.at[pl.ds(s * RPT, RPT), pl.ds(col, 32)])

        if p == 0:
            @pl.when(c == 0)
            def _():
                pltpu.sync_copy(cnta.at[pl.ds(s * RPT, RPT)],
                                pcnt.at[pl.ds(s * RPT, RPT)])
        plsc.subcore_barrier()


def _agg_sup(tables, edges, zrows, z1, ones):
    mesh = plsc.VectorSubcoreMesh(core_axis_name="c", subcore_axis_name="s",
                                  num_cores=NC, num_subcores=NS)
    fn = pl.kernel(
        _agg_sup_body,
        out_type=[
            jax.ShapeDtypeStruct((RP, H), jnp.float32),
            jax.ShapeDtypeStruct((RP,), jnp.float32),
        ],
        mesh=mesh,
        scratch_types=[
            pltpu.VMEM_SHARED((RP, 32), jnp.float32),
            pltpu.VMEM_SHARED((RP,), jnp.float32),
            pltpu.VMEM((4, 2, CH), jnp.int32),
            pltpu.VMEM((4, CH, 32), jnp.float32),
            pltpu.VMEM((CH,), jnp.float32),
            pltpu.VMEM((CH,), jnp.float32),
            pltpu.SemaphoreType.DMA((4,)),
            pltpu.SemaphoreType.DMA((4,)),
            pltpu.SemaphoreType.DMA((4,)),
        ],
        compiler_params=pltpu.CompilerParams(use_tc_tiling_on_sc=False),
    )
    return fn(*tables, edges, zrows, z1, ones)


def _agg_st_body(table, edges, zrows, z1, ones,
                 psum, pcnt,
                 acc, cnta, ibuf, rbuf, onev, zflat, isem, gsem, ssem):
    c = lax.axis_index("c")
    s = lax.axis_index("s")
    wid = c * NS + s
    nch = E2P // (NW * CH)
    row0 = wid * nch

    pltpu.sync_copy(ones, onev)
    pltpu.sync_copy(z1, zflat)
    pltpu.sync_copy(zrows, rbuf.at[0])
    _zero_rows(rbuf.at[0], zflat, acc, cnta, s * RWT, RWT, CH, True)
    plsc.subcore_barrier()
    _seg_pipeline(edges, row0, nch, table, acc, cnta, ibuf, rbuf, onev,
                  isem, gsem, ssem, True)
    plsc.subcore_barrier()
    pltpu.sync_copy(acc.at[pl.ds(s * RWT, RWT)],
                    psum.at[c, pl.ds(s * RWT, RWT)])
    pltpu.sync_copy(cnta.at[pl.ds(s * RWT, RWT)],
                    pcnt.at[c, pl.ds(s * RWT, RWT)])


def _agg_st(table, edges, zrows, z1, ones):
    mesh = plsc.VectorSubcoreMesh(core_axis_name="c", subcore_axis_name="s",
                                  num_cores=NC, num_subcores=NS)
    fn = pl.kernel(
        _agg_st_body,
        out_type=[
            jax.ShapeDtypeStruct((NC, RW, H), jnp.float32),
            jax.ShapeDtypeStruct((NC, RW), jnp.float32),
        ],
        mesh=mesh,
        scratch_types=[
            pltpu.VMEM_SHARED((RW, H), jnp.float32),
            pltpu.VMEM_SHARED((RW,), jnp.float32),
            pltpu.VMEM((4, 2, CH), jnp.int32),
            pltpu.VMEM((2, CH, H), jnp.float32),
            pltpu.VMEM((CH,), jnp.float32),
            pltpu.VMEM((CH,), jnp.float32),
            pltpu.SemaphoreType.DMA((4,)),
            pltpu.SemaphoreType.DMA((2,)),
            pltpu.SemaphoreType.DMA((2,)),
        ],
    )
    return fn(table, edges, zrows, z1, ones)


def _pad_edges(ei, n_edges, n_pad_total, n_src, n_dst_real, n_garbage):
    src, dst = ei[0], ei[1]
    npad = n_pad_total - n_edges
    pad_i = jnp.arange(npad, dtype=jnp.int32)
    src = jnp.concatenate([src, pad_i % n_src])
    dst = jnp.concatenate([dst, n_dst_real + pad_i % n_garbage])
    return jnp.stack([src.reshape(-1, CH), dst.reshape(-1, CH)], axis=1)


def kernel(x_supplier, x_product, x_warehouse, ei_supplies, ei_stored,
           W_sup, b_sup, W_prod, b_prod, W_wh, b_wh,
           l0_sup_Wl, l0_sup_bl, l0_sup_Wr, l0_st_Wl, l0_st_bl, l0_st_Wr,
           l1_sup_Wl, l1_sup_bl, l1_sup_Wr, l1_st_Wl, l1_st_bl, l1_st_Wr):
    h_sup = _proj(x_supplier, W_sup, b_sup)
    h_prod0 = _proj(x_product, W_prod, b_prod)
    h_wh0 = _proj(x_warehouse, W_wh, b_wh)

    edgesA = _pad_edges(ei_supplies, ei_supplies.shape[1], E1P,
                        N_SUP, N_PROD, RP - N_PROD)
    edgesB = _pad_edges(ei_stored, ei_stored.shape[1], E2P,
                        N_PROD, N_WH, RW - N_WH)

    zrowsA = jnp.zeros((CH, 32), jnp.float32)
    zrowsB = jnp.zeros((CH, H), jnp.float32)
    z1 = jnp.zeros((CH,), jnp.float32)
    ones = jnp.ones((CH,), jnp.float32)

    NREP = 4
    tables = tuple(jnp.tile(h_sup[:, f * 32:(f + 1) * 32], (NREP, 1))
                   for f in range(4))
    nch_a = E1P // (NS * CH)
    owner = (jnp.arange(E1P // CH, dtype=jnp.int32) // nch_a) % NREP
    edgesA = edgesA.at[:, 0, :].add(owner[:, None] * N_SUP)
    psumA, pcntA = _agg_sup(tables, edgesA, zrowsA, z1, ones)
    cA = pcntA.reshape(RP, 1)

    sumB0, cntB0 = _agg_st(h_prod0, edgesB, zrowsB, z1, ones)

    h_prod1 = _combine1(psumA, cA, h_prod0,
                        l0_sup_Wl, l0_sup_Wr, l0_sup_bl)
    h_wh1 = _combine(sumB0[0], sumB0[1], cntB0[0].reshape(RW, 1),
                     cntB0[1].reshape(RW, 1), h_wh0,
                     l0_st_Wl, l0_st_Wr, l0_st_bl)

    sumB1, cntB1 = _agg_st(h_prod1, edgesB, zrowsB, z1, ones)

    h_prod2 = _combine1(psumA, cA, h_prod1,
                        l1_sup_Wl, l1_sup_Wr, l1_sup_bl)
    h_wh2 = _combine(sumB1[0], sumB1[1], cntB1[0].reshape(RW, 1),
                     cntB1[1].reshape(RW, 1), h_wh1,
                     l1_st_Wl, l1_st_Wr, l1_st_bl)

    return (h_sup, h_prod2, h_wh2)


# speedup vs baseline: 1.1530x; 1.1530x over previous
import functools

import jax
import jax.numpy as jnp
from jax import lax
from jax.experimental import pallas as pl
from jax.experimental.pallas import tpu as pltpu
from jax.experimental.pallas import tpu_sc as plsc

N_SUP = 10000
N_PROD = 50000
N_WH = 10000
H = 128

NC = 2
NS = 16
NW = NC * NS

CH = 128
CHA = 64

RP = 50048
RPT = RP // NS
RW = 10240
RWT = RW // NS
TSP = 10240

E1P = 327680
E2P = 294912


def _zero_rows(zbuf, zflat, acc, cnta, base, nrows, zr, with_cnt):
    full, tail = nrows // zr, nrows % zr
    for z in range(full):
        pltpu.sync_copy(zbuf, acc.at[pl.ds(base + z * zr, zr)])
        if with_cnt:
            pltpu.sync_copy(zflat, cnta.at[pl.ds(base + z * zr, zr)])
    if tail:
        pltpu.sync_copy(zbuf.at[pl.ds(0, tail)],
                        acc.at[pl.ds(base + full * zr, tail)])
        if with_cnt:
            pltpu.sync_copy(zflat.at[pl.ds(0, tail)],
                            cnta.at[pl.ds(base + full * zr, tail)])


def _proj_body(x_ref, w_ref, b_ref, o_ref):
    o_ref[...] = jnp.maximum(
        jnp.dot(x_ref[...], w_ref[...], preferred_element_type=jnp.float32)
        + b_ref[...], 0.0)


def _proj(x, W, b, bm=512):
    n, d = x.shape
    h = W.shape[1]
    return pl.pallas_call(
        _proj_body,
        grid=(pl.cdiv(n, bm),),
        in_specs=[
            pl.BlockSpec((bm, d), lambda i: (i, 0)),
            pl.BlockSpec((d, h), lambda i: (0, 0)),
            pl.BlockSpec((1, h), lambda i: (0, 0)),
        ],
        out_specs=pl.BlockSpec((bm, h), lambda i: (i, 0)),
        out_shape=jax.ShapeDtypeStruct((n, h), jnp.float32),
    )(x, W, b.reshape(1, h))


def _combine_body(p0_ref, p1_ref, c0_ref, c1_ref, x_ref, wl_ref, wr_ref,
                  b_ref, o_ref):
    cnt = jnp.maximum(c0_ref[...] + c1_ref[...], 1.0)
    m = (p0_ref[...] + p1_ref[...]) / cnt
    o_ref[...] = jnp.maximum(
        jnp.dot(m, wl_ref[...], preferred_element_type=jnp.float32)
        + jnp.dot(x_ref[...], wr_ref[...], preferred_element_type=jnp.float32)
        + b_ref[...], 0.0)


def _combine1_body(p_ref, c_ref, x_ref, wl_ref, wr_ref, b_ref, o_ref):
    cnt = jnp.maximum(c_ref[...], 1.0)
    m = p_ref[...] / cnt
    o_ref[...] = jnp.maximum(
        jnp.dot(m, wl_ref[...], preferred_element_type=jnp.float32)
        + jnp.dot(x_ref[...], wr_ref[...], preferred_element_type=jnp.float32)
        + b_ref[...], 0.0)


def _combine1(p, cnt, x, Wl, Wr, b, bm=512):
    n = x.shape[0]
    return pl.pallas_call(
        _combine1_body,
        grid=(pl.cdiv(n, bm),),
        in_specs=[
            pl.BlockSpec((bm, H), lambda i: (i, 0)),
            pl.BlockSpec((bm, 1), lambda i: (i, 0)),
            pl.BlockSpec((bm, H), lambda i: (i, 0)),
            pl.BlockSpec((H, H), lambda i: (0, 0)),
            pl.BlockSpec((H, H), lambda i: (0, 0)),
            pl.BlockSpec((1, H), lambda i: (0, 0)),
        ],
        out_specs=pl.BlockSpec((bm, H), lambda i: (i, 0)),
        out_shape=jax.ShapeDtypeStruct((n, H), jnp.float32),
    )(p, cnt, x, Wl, Wr, b.reshape(1, H))


def _combine(p0, p1, c0, c1, x, Wl, Wr, b, bm=512):
    n = x.shape[0]
    return pl.pallas_call(
        _combine_body,
        grid=(pl.cdiv(n, bm),),
        in_specs=[
            pl.BlockSpec((bm, H), lambda i: (i, 0)),
            pl.BlockSpec((bm, H), lambda i: (i, 0)),
            pl.BlockSpec((bm, 1), lambda i: (i, 0)),
            pl.BlockSpec((bm, 1), lambda i: (i, 0)),
            pl.BlockSpec((bm, H), lambda i: (i, 0)),
            pl.BlockSpec((H, H), lambda i: (0, 0)),
            pl.BlockSpec((H, H), lambda i: (0, 0)),
            pl.BlockSpec((1, H), lambda i: (0, 0)),
        ],
        out_specs=pl.BlockSpec((bm, H), lambda i: (i, 0)),
        out_shape=jax.ShapeDtypeStruct((n, H), jnp.float32),
    )(p0, p1, c0, c1, x, Wl, Wr, b.reshape(1, H))


def _seg_pipeline(edges, row0, nch, table, acc, cnta, ibuf, rbuf, onev,
                  isem, gsem, ssem, with_cnt):
    def idx_load(j, sl):
        pltpu.async_copy(edges.at[row0 + j], ibuf.at[sl], isem.at[sl])

    def idx_wait(sl):
        pltpu.make_async_copy(edges.at[0], ibuf.at[sl], isem.at[sl]).wait()

    def gather(sl, rs):
        pltpu.async_copy(table.at[ibuf.at[sl, 0]], rbuf.at[rs], gsem.at[rs])

    def gather_wait(sl, rs):
        pltpu.make_async_copy(table.at[ibuf.at[sl, 0]], rbuf.at[rs],
                              gsem.at[rs]).wait()

    def scat(sl, rs):
        pltpu.async_copy(rbuf.at[rs], acc.at[ibuf.at[sl, 1]], ssem.at[rs],
                         add=True)
        if with_cnt:
            pltpu.async_copy(onev, cnta.at[ibuf.at[sl, 1]], ssem.at[rs],
                             add=True)

    def scat_wait(sl, rs):
        pltpu.make_async_copy(rbuf.at[rs], acc.at[ibuf.at[sl, 1]],
                              ssem.at[rs]).wait()
        if with_cnt:
            pltpu.make_async_copy(onev, cnta.at[ibuf.at[sl, 1]],
                                  ssem.at[rs]).wait()

    def step(j, u, prefetch):
        rs = u % 2
        scat_wait((u - 2) % 4, rs)
        if prefetch:
            idx_load(j + 2, (u + 2) % 4)
        idx_wait(u)
        gather(u, rs)
        gather_wait((u - 1) % 4, 1 - rs)
        scat((u - 1) % 4, 1 - rs)

    for sl in range(4):
        idx_load(sl, sl)
    idx_wait(0)
    gather(0, 0)
    idx_wait(1)
    gather(1, 1)
    gather_wait(0, 0)
    scat(0, 0)

    def grp(m, carry):
        j0 = 2 + 4 * m
        step(j0, 2, True)
        step(j0 + 1, 3, True)
        step(j0 + 2, 0, True)
        step(j0 + 3, 1, True)
        return carry

    lax.fori_loop(0, (nch - 4) // 4, grp, 0)
    step(nch - 2, 2, False)
    step(nch - 1, 3, False)
    gather_wait(3, 1)
    scat(3, 1)
    scat_wait(2, 0)
    scat_wait(3, 1)


def _seg_pipeline4(edges, row0, nch, table, acc, cnta, ibuf, rbuf, onev,
                   isem, gsem, ssem, with_cnt):
    def idx_load(j, sl):
        pltpu.async_copy(edges.at[row0 + j], ibuf.at[sl], isem.at[sl])

    def idx_wait(sl):
        pltpu.make_async_copy(edges.at[0], ibuf.at[sl], isem.at[sl]).wait()

    def gather(sl):
        pltpu.async_copy(table.at[ibuf.at[sl, 0]], rbuf.at[sl], gsem.at[sl])

    def gather_wait(sl):
        pltpu.make_async_copy(table.at[ibuf.at[sl, 0]], rbuf.at[sl],
                              gsem.at[sl]).wait()

    def scat(sl):
        pltpu.async_copy(rbuf.at[sl], acc.at[ibuf.at[sl, 1]], ssem.at[sl],
                         add=True)
        if with_cnt:
            pltpu.async_copy(onev, cnta.at[ibuf.at[sl, 1]], ssem.at[sl],
                             add=True)

    def scat_wait(sl):
        pltpu.make_async_copy(rbuf.at[sl], acc.at[ibuf.at[sl, 1]],
                              ssem.at[sl]).wait()
        if with_cnt:
            pltpu.make_async_copy(onev, cnta.at[ibuf.at[sl, 1]],
                                  ssem.at[sl]).wait()

    def step(j, u, first=False, prefetch=True, gather_next=True):
        if not first:
            scat_wait((u + 2) % 4)
        if prefetch:
            idx_load(j + 2, (u + 2) % 4)
        if gather_next:
            idx_wait((u + 1) % 4)
            gather((u + 1) % 4)
        gather_wait(u)
        scat(u)

    idx_load(0, 0)
    idx_load(1, 1)
    idx_wait(0)
    gather(0)
    step(0, 0, first=True)
    step(1, 1, first=True)

    def grp(m, carry):
        j0 = 2 + 4 * m
        step(j0, 2)
        step(j0 + 1, 3)
        step(j0 + 2, 0)
        step(j0 + 3, 1)
        return carry

    lax.fori_loop(0, (nch - 4) // 4, grp, 0)
    step(nch - 2, 2, prefetch=False)
    step(nch - 1, 3, prefetch=False, gather_next=False)
    scat_wait(2)
    scat_wait(3)


def _agg_sup_body(t0, t1, t2, t3, edges, zrows, z1, ones,
                  psum, pcnt,
                  acc, cnta, tbl, ibuf, rbuf, onev, zflat, isem, gsem, ssem):
    c = lax.axis_index("c")
    s = lax.axis_index("s")
    nch = E1P // (NS * CHA)
    row0 = s * nch
    tpt = TSP // NS

    pltpu.sync_copy(ones.at[pl.ds(0, CHA)], onev)
    pltpu.sync_copy(z1.at[pl.ds(0, CHA)], zflat)

    for p in range(2):
        for cc in range(NC):
            tf = (t0, t1, t2, t3)[2 * cc + p]

            @pl.when(c == cc)
            def _():
                for i in range(tpt // CHA):
                    r = s * tpt + i * CHA
                    pltpu.sync_copy(tf.at[pl.ds(r, CHA)], rbuf.at[0])
                    pltpu.sync_copy(rbuf.at[0], tbl.at[pl.ds(r, CHA)])

        pltpu.sync_copy(zrows, rbuf.at[0])
        _zero_rows(rbuf.at[0], zflat, acc, cnta, s * RPT, RPT, CHA, p == 0)
        plsc.subcore_barrier()
        for cc in range(NC):
            wc = p == 0 and cc == 0

            @pl.when(c == cc)
            def _():
                _seg_pipeline(edges, row0, nch, tbl, acc, cnta, ibuf, rbuf,
                              onev, isem, gsem, ssem, wc)

        plsc.subcore_barrier()
        for cc in range(NC):
            col = (2 * cc + p) * 32

            @pl.when(c == cc)
            def _():
                pltpu.sync_copy(
                    acc.at[pl.ds(s * RPT, RPT)],
                    psum.at[pl.ds(s * RPT, RPT), pl.ds(col, 32)])

        if p == 0:
            @pl.when(c == 0)
            def _():
                pltpu.sync_copy(cnta.at[pl.ds(s * RPT, RPT)],
                                pcnt.at[pl.ds(s * RPT, RPT)])
        plsc.subcore_barrier()


def _agg_sup(tables, edges, zrows, z1, ones):
    mesh = plsc.VectorSubcoreMesh(core_axis_name="c", subcore_axis_name="s",
                                  num_cores=NC, num_subcores=NS)
    fn = pl.kernel(
        _agg_sup_body,
        out_type=[
            jax.ShapeDtypeStruct((RP, H), jnp.float32),
            jax.ShapeDtypeStruct((RP,), jnp.float32),
        ],
        mesh=mesh,
        scratch_types=[
            pltpu.VMEM_SHARED((RP, 32), jnp.float32),
            pltpu.VMEM_SHARED((RP,), jnp.float32),
            pltpu.VMEM_SHARED((TSP, 32), jnp.float32),
            pltpu.VMEM((4, 2, CHA), jnp.int32),
            pltpu.VMEM((2, CHA, 32), jnp.float32),
            pltpu.VMEM((CHA,), jnp.float32),
            pltpu.VMEM((CHA,), jnp.float32),
            pltpu.SemaphoreType.DMA((4,)),
            pltpu.SemaphoreType.DMA((2,)),
            pltpu.SemaphoreType.DMA((2,)),
        ],
        compiler_params=pltpu.CompilerParams(use_tc_tiling_on_sc=False),
    )
    return fn(*tables, edges, zrows, z1, ones)


def _agg_st_body(table, edges, zrows, z1, ones,
                 psum, pcnt,
                 acc, cnta, ibuf, rbuf, onev, zflat, isem, gsem, ssem):
    c = lax.axis_index("c")
    s = lax.axis_index("s")
    wid = c * NS + s
    nch = E2P // (NW * CH)
    row0 = wid * nch

    pltpu.sync_copy(ones, onev)
    pltpu.sync_copy(z1, zflat)
    pltpu.sync_copy(zrows, rbuf.at[0])
    _zero_rows(rbuf.at[0], zflat, acc, cnta, s * RWT, RWT, CH, True)
    plsc.subcore_barrier()
    _seg_pipeline(edges, row0, nch, table, acc, cnta, ibuf, rbuf, onev,
                  isem, gsem, ssem, True)
    plsc.subcore_barrier()
    pltpu.sync_copy(acc.at[pl.ds(s * RWT, RWT)],
                    psum.at[c, pl.ds(s * RWT, RWT)])
    pltpu.sync_copy(cnta.at[pl.ds(s * RWT, RWT)],
                    pcnt.at[c, pl.ds(s * RWT, RWT)])


def _agg_st(table, edges, zrows, z1, ones):
    mesh = plsc.VectorSubcoreMesh(core_axis_name="c", subcore_axis_name="s",
                                  num_cores=NC, num_subcores=NS)
    fn = pl.kernel(
        _agg_st_body,
        out_type=[
            jax.ShapeDtypeStruct((NC, RW, H), jnp.float32),
            jax.ShapeDtypeStruct((NC, RW), jnp.float32),
        ],
        mesh=mesh,
        scratch_types=[
            pltpu.VMEM_SHARED((RW, H), jnp.float32),
            pltpu.VMEM_SHARED((RW,), jnp.float32),
            pltpu.VMEM((4, 2, CH), jnp.int32),
            pltpu.VMEM((2, CH, H), jnp.float32),
            pltpu.VMEM((CH,), jnp.float32),
            pltpu.VMEM((CH,), jnp.float32),
            pltpu.SemaphoreType.DMA((4,)),
            pltpu.SemaphoreType.DMA((2,)),
            pltpu.SemaphoreType.DMA((2,)),
        ],
    )
    return fn(table, edges, zrows, z1, ones)


def _pad_edges(ei, n_edges, n_pad_total, n_src, n_dst_real, n_garbage, ch):
    src, dst = ei[0], ei[1]
    npad = n_pad_total - n_edges
    pad_i = jnp.arange(npad, dtype=jnp.int32)
    src = jnp.concatenate([src, pad_i % n_src])
    dst = jnp.concatenate([dst, n_dst_real + pad_i % n_garbage])
    return jnp.stack([src.reshape(-1, ch), dst.reshape(-1, ch)], axis=1)


def kernel(x_supplier, x_product, x_warehouse, ei_supplies, ei_stored,
           W_sup, b_sup, W_prod, b_prod, W_wh, b_wh,
           l0_sup_Wl, l0_sup_bl, l0_sup_Wr, l0_st_Wl, l0_st_bl, l0_st_Wr,
           l1_sup_Wl, l1_sup_bl, l1_sup_Wr, l1_st_Wl, l1_st_bl, l1_st_Wr):
    h_sup = _proj(x_supplier, W_sup, b_sup)
    h_prod0 = _proj(x_product, W_prod, b_prod)
    h_wh0 = _proj(x_warehouse, W_wh, b_wh)

    edgesA = _pad_edges(ei_supplies, ei_supplies.shape[1], E1P,
                        N_SUP, N_PROD, RP - N_PROD, CHA)
    edgesB = _pad_edges(ei_stored, ei_stored.shape[1], E2P,
                        N_PROD, N_WH, RW - N_WH, CH)

    zrowsA = jnp.zeros((CHA, 32), jnp.float32)
    zrowsB = jnp.zeros((CH, H), jnp.float32)
    z1 = jnp.zeros((CH,), jnp.float32)
    ones = jnp.ones((CH,), jnp.float32)

    tables = tuple(
        jnp.pad(h_sup[:, f * 32:(f + 1) * 32], ((0, TSP - N_SUP), (0, 0)))
        for f in range(4))
    psumA, pcntA = _agg_sup(tables, edgesA, zrowsA, z1, ones)
    cA = pcntA.reshape(RP, 1)

    sumB0, cntB0 = _agg_st(h_prod0, edgesB, zrowsB, z1, ones)

    h_prod1 = _combine1(psumA, cA, h_prod0,
                        l0_sup_Wl, l0_sup_Wr, l0_sup_bl)
    h_wh1 = _combine(sumB0[0], sumB0[1], cntB0[0].reshape(RW, 1),
                     cntB0[1].reshape(RW, 1), h_wh0,
                     l0_st_Wl, l0_st_Wr, l0_st_bl)

    sumB1, cntB1 = _agg_st(h_prod1, edgesB, zrowsB, z1, ones)

    h_prod2 = _combine1(psumA, cA, h_prod1,
                        l1_sup_Wl, l1_sup_Wr, l1_sup_bl)
    h_wh2 = _combine(sumB1[0], sumB1[1], cntB1[0].reshape(RW, 1),
                     cntB1[1].reshape(RW, 1), h_wh1,
                     l1_st_Wl, l1_st_Wr, l1_st_bl)

    return (h_sup, h_prod2, h_wh2)


# speedup vs baseline: 1.1880x; 1.0303x over previous
import functools

import jax
import jax.numpy as jnp
from jax import lax
from jax.experimental import pallas as pl
from jax.experimental.pallas import tpu as pltpu
from jax.experimental.pallas import tpu_sc as plsc

N_SUP = 10000
N_PROD = 50000
N_WH = 10000
H = 128

NC = 2
NS = 16
NW = NC * NS

CH = 128

RP = 50048
RPT = RP // NS
RW = 10240
RWT = RW // NS

E1P = 327680
E2P = 294912


def _zero_rows(zbuf, zflat, acc, cnta, base, nrows, zr, with_cnt):
    full, tail = nrows // zr, nrows % zr
    for z in range(full):
        pltpu.sync_copy(zbuf, acc.at[pl.ds(base + z * zr, zr)])
        if with_cnt:
            pltpu.sync_copy(zflat, cnta.at[pl.ds(base + z * zr, zr)])
    if tail:
        pltpu.sync_copy(zbuf.at[pl.ds(0, tail)],
                        acc.at[pl.ds(base + full * zr, tail)])
        if with_cnt:
            pltpu.sync_copy(zflat.at[pl.ds(0, tail)],
                            cnta.at[pl.ds(base + full * zr, tail)])


def _proj_body(x_ref, w_ref, b_ref, o_ref):
    o_ref[...] = jnp.maximum(
        jnp.dot(x_ref[...], w_ref[...], preferred_element_type=jnp.float32)
        + b_ref[...], 0.0)


def _proj(x, W, b, bm=512):
    n, d = x.shape
    h = W.shape[1]
    return pl.pallas_call(
        _proj_body,
        grid=(pl.cdiv(n, bm),),
        in_specs=[
            pl.BlockSpec((bm, d), lambda i: (i, 0)),
            pl.BlockSpec((d, h), lambda i: (0, 0)),
            pl.BlockSpec((1, h), lambda i: (0, 0)),
        ],
        out_specs=pl.BlockSpec((bm, h), lambda i: (i, 0)),
        out_shape=jax.ShapeDtypeStruct((n, h), jnp.float32),
    )(x, W, b.reshape(1, h))


def _combine_body(p0_ref, p1_ref, c0_ref, c1_ref, x_ref, wl_ref, wr_ref,
                  b_ref, o_ref):
    cnt = jnp.maximum(c0_ref[...] + c1_ref[...], 1.0)
    m = (p0_ref[...] + p1_ref[...]) / cnt
    o_ref[...] = jnp.maximum(
        jnp.dot(m, wl_ref[...], preferred_element_type=jnp.float32)
        + jnp.dot(x_ref[...], wr_ref[...], preferred_element_type=jnp.float32)
        + b_ref[...], 0.0)


def _combine1_body(p_ref, c_ref, x_ref, wl_ref, wr_ref, b_ref, o_ref):
    cnt = jnp.maximum(c_ref[...], 1.0)
    m = p_ref[...] / cnt
    o_ref[...] = jnp.maximum(
        jnp.dot(m, wl_ref[...], preferred_element_type=jnp.float32)
        + jnp.dot(x_ref[...], wr_ref[...], preferred_element_type=jnp.float32)
        + b_ref[...], 0.0)


def _combine1(p, cnt, x, Wl, Wr, b, bm=512):
    n = x.shape[0]
    return pl.pallas_call(
        _combine1_body,
        grid=(pl.cdiv(n, bm),),
        in_specs=[
            pl.BlockSpec((bm, H), lambda i: (i, 0)),
            pl.BlockSpec((bm, 1), lambda i: (i, 0)),
            pl.BlockSpec((bm, H), lambda i: (i, 0)),
            pl.BlockSpec((H, H), lambda i: (0, 0)),
            pl.BlockSpec((H, H), lambda i: (0, 0)),
            pl.BlockSpec((1, H), lambda i: (0, 0)),
        ],
        out_specs=pl.BlockSpec((bm, H), lambda i: (i, 0)),
        out_shape=jax.ShapeDtypeStruct((n, H), jnp.float32),
    )(p, cnt, x, Wl, Wr, b.reshape(1, H))


def _combine(p0, p1, c0, c1, x, Wl, Wr, b, bm=512):
    n = x.shape[0]
    return pl.pallas_call(
        _combine_body,
        grid=(pl.cdiv(n, bm),),
        in_specs=[
            pl.BlockSpec((bm, H), lambda i: (i, 0)),
            pl.BlockSpec((bm, H), lambda i: (i, 0)),
            pl.BlockSpec((bm, 1), lambda i: (i, 0)),
            pl.BlockSpec((bm, 1), lambda i: (i, 0)),
            pl.BlockSpec((bm, H), lambda i: (i, 0)),
            pl.BlockSpec((H, H), lambda i: (0, 0)),
            pl.BlockSpec((H, H), lambda i: (0, 0)),
            pl.BlockSpec((1, H), lambda i: (0, 0)),
        ],
        out_specs=pl.BlockSpec((bm, H), lambda i: (i, 0)),
        out_shape=jax.ShapeDtypeStruct((n, H), jnp.float32),
    )(p0, p1, c0, c1, x, Wl, Wr, b.reshape(1, H))


def _seg_pipeline(edges, row0, nch, table, acc, cnta, ibuf, rbuf, onev,
                  isem, gsem, ssem, with_cnt):
    def idx_load(j, sl):
        pltpu.async_copy(edges.at[row0 + j], ibuf.at[sl], isem.at[sl])

    def idx_wait(sl):
        pltpu.make_async_copy(edges.at[0], ibuf.at[sl], isem.at[sl]).wait()

    def gather(sl, rs):
        pltpu.async_copy(table.at[ibuf.at[sl, 0]], rbuf.at[rs], gsem.at[rs])

    def gather_wait(sl, rs):
        pltpu.make_async_copy(table.at[ibuf.at[sl, 0]], rbuf.at[rs],
                              gsem.at[rs]).wait()

    def scat(sl, rs):
        pltpu.async_copy(rbuf.at[rs], acc.at[ibuf.at[sl, 1]], ssem.at[rs],
                         add=True)
        if with_cnt:
            pltpu.async_copy(onev, cnta.at[ibuf.at[sl, 1]], ssem.at[rs],
                             add=True)

    def scat_wait(sl, rs):
        pltpu.make_async_copy(rbuf.at[rs], acc.at[ibuf.at[sl, 1]],
                              ssem.at[rs]).wait()
        if with_cnt:
            pltpu.make_async_copy(onev, cnta.at[ibuf.at[sl, 1]],
                                  ssem.at[rs]).wait()

    def step(j, u, prefetch):
        rs = u % 2
        scat_wait((u - 2) % 4, rs)
        if prefetch:
            idx_load(j + 2, (u + 2) % 4)
        idx_wait(u)
        gather(u, rs)
        gather_wait((u - 1) % 4, 1 - rs)
        scat((u - 1) % 4, 1 - rs)

    for sl in range(4):
        idx_load(sl, sl)
    idx_wait(0)
    gather(0, 0)
    idx_wait(1)
    gather(1, 1)
    gather_wait(0, 0)
    scat(0, 0)

    def grp(m, carry):
        j0 = 2 + 4 * m
        step(j0, 2, True)
        step(j0 + 1, 3, True)
        step(j0 + 2, 0, True)
        step(j0 + 3, 1, True)
        return carry

    lax.fori_loop(0, (nch - 4) // 4, grp, 0)
    step(nch - 2, 2, False)
    step(nch - 1, 3, False)
    gather_wait(3, 1)
    scat(3, 1)
    scat_wait(2, 0)
    scat_wait(3, 1)


def _seg_pipeline4(edges, row0, nch, table, acc, cnta, ibuf, rbuf, onev,
                   isem, gsem, ssem, with_cnt):
    def idx_load(j, sl):
        pltpu.async_copy(edges.at[row0 + j], ibuf.at[sl], isem.at[sl])

    def idx_wait(sl):
        pltpu.make_async_copy(edges.at[0], ibuf.at[sl], isem.at[sl]).wait()

    def gather(sl):
        pltpu.async_copy(table.at[ibuf.at[sl, 0]], rbuf.at[sl], gsem.at[sl])

    def gather_wait(sl):
        pltpu.make_async_copy(table.at[ibuf.at[sl, 0]], rbuf.at[sl],
                              gsem.at[sl]).wait()

    def scat(sl):
        pltpu.async_copy(rbuf.at[sl], acc.at[ibuf.at[sl, 1]], ssem.at[sl],
                         add=True)
        if with_cnt:
            pltpu.async_copy(onev, cnta.at[ibuf.at[sl, 1]], ssem.at[sl],
                             add=True)

    def scat_wait(sl):
        pltpu.make_async_copy(rbuf.at[sl], acc.at[ibuf.at[sl, 1]],
                              ssem.at[sl]).wait()
        if with_cnt:
            pltpu.make_async_copy(onev, cnta.at[ibuf.at[sl, 1]],
                                  ssem.at[sl]).wait()

    def step(j, u, first=False, prefetch=True, gather_next=True):
        if not first:
            scat_wait((u + 2) % 4)
        if prefetch:
            idx_load(j + 2, (u + 2) % 4)
        if gather_next:
            idx_wait((u + 1) % 4)
            gather((u + 1) % 4)
        gather_wait(u)
        scat(u)

    idx_load(0, 0)
    idx_load(1, 1)
    idx_wait(0)
    gather(0)
    step(0, 0, first=True)
    step(1, 1, first=True)

    def grp(m, carry):
        j0 = 2 + 4 * m
        step(j0, 2)
        step(j0 + 1, 3)
        step(j0 + 2, 0)
        step(j0 + 3, 1)
        return carry

    lax.fori_loop(0, (nch - 4) // 4, grp, 0)
    step(nch - 2, 2, prefetch=False)
    step(nch - 1, 3, prefetch=False, gather_next=False)
    scat_wait(2)
    scat_wait(3)


def _agg_sup_body(t0, t1, t2, t3, edges, zrows, z1, ones,
                  psum, pcnt,
                  acc, cnta, ibuf, rbuf, onev, zflat, isem, gsem, ssem):
    c = lax.axis_index("c")
    s = lax.axis_index("s")
    nch = E1P // (NS * CH)
    row0 = s * nch

    pltpu.sync_copy(ones, onev)
    pltpu.sync_copy(z1, zflat)

    for p in range(2):
        pltpu.sync_copy(zrows, rbuf.at[0])
        _zero_rows(rbuf.at[0], zflat, acc, cnta, s * RPT, RPT, CH, p == 0)
        plsc.subcore_barrier()
        for cc in range(NC):
            tf = (t0, t1, t2, t3)[2 * cc + p]
            wc = p == 0 and cc == 0

            @pl.when(c == cc)
            def _():
                _seg_pipeline4(edges, row0, nch, tf, acc, cnta, ibuf, rbuf,
                               onev, isem, gsem, ssem, wc)

        plsc.subcore_barrier()
        for cc in range(NC):
            col = (2 * cc + p) * 32

            @pl.when(c == cc)
            def _():
                pltpu.sync_copy(
                    acc.at[pl.ds(s * RPT, RPT)],
                    psum.at[pl.ds(s * RPT, RPT), pl.ds(col, 32)])

        if p == 0:
            @pl.when(c == 0)
            def _():
                pltpu.sync_copy(cnta.at[pl.ds(s * RPT, RPT)],
                                pcnt.at[pl.ds(s * RPT, RPT)])
        plsc.subcore_barrier()


def _agg_sup(tables, edges, zrows, z1, ones):
    mesh = plsc.VectorSubcoreMesh(core_axis_name="c", subcore_axis_name="s",
                                  num_cores=NC, num_subcores=NS)
    fn = pl.kernel(
        _agg_sup_body,
        out_type=[
            jax.ShapeDtypeStruct((RP, H), jnp.float32),
            jax.ShapeDtypeStruct((RP,), jnp.float32),
        ],
        mesh=mesh,
        scratch_types=[
            pltpu.VMEM_SHARED((RP, 32), jnp.float32),
            pltpu.VMEM_SHARED((RP,), jnp.float32),
            pltpu.VMEM((4, 2, CH), jnp.int32),
            pltpu.VMEM((4, CH, 32), jnp.float32),
            pltpu.VMEM((CH,), jnp.float32),
            pltpu.VMEM((CH,), jnp.float32),
            pltpu.SemaphoreType.DMA((4,)),
            pltpu.SemaphoreType.DMA((4,)),
            pltpu.SemaphoreType.DMA((4,)),
        ],
        compiler_params=pltpu.CompilerParams(use_tc_tiling_on_sc=False),
    )
    return fn(*tables, edges, zrows, z1, ones)


def _agg_st_body(table, edges, zrows, z1, ones,
                 psum, pcnt,
                 acc, cnta, ibuf, rbuf, onev, zflat, isem, gsem, ssem):
    c = lax.axis_index("c")
    s = lax.axis_index("s")
    wid = c * NS + s
    nch = E2P // (NW * CH)
    row0 = wid * nch

    pltpu.sync_copy(ones, onev)
    pltpu.sync_copy(z1, zflat)
    pltpu.sync_copy(zrows, rbuf.at[0])
    _zero_rows(rbuf.at[0], zflat, acc, cnta, s * RWT, RWT, CH, True)
    plsc.subcore_barrier()
    _seg_pipeline(edges, row0, nch, table, acc, cnta, ibuf, rbuf, onev,
                  isem, gsem, ssem, True)
    plsc.subcore_barrier()
    pltpu.sync_copy(acc.at[pl.ds(s * RWT, RWT)],
                    psum.at[c, pl.ds(s * RWT, RWT)])
    pltpu.sync_copy(cnta.at[pl.ds(s * RWT, RWT)],
                    pcnt.at[c, pl.ds(s * RWT, RWT)])


def _agg_st(table, edges, zrows, z1, ones):
    mesh = plsc.VectorSubcoreMesh(core_axis_name="c", subcore_axis_name="s",
                                  num_cores=NC, num_subcores=NS)
    fn = pl.kernel(
        _agg_st_body,
        out_type=[
            jax.ShapeDtypeStruct((NC, RW, H), jnp.float32),
            jax.ShapeDtypeStruct((NC, RW), jnp.float32),
        ],
        mesh=mesh,
        scratch_types=[
            pltpu.VMEM_SHARED((RW, H), jnp.float32),
            pltpu.VMEM_SHARED((RW,), jnp.float32),
            pltpu.VMEM((4, 2, CH), jnp.int32),
            pltpu.VMEM((2, CH, H), jnp.float32),
            pltpu.VMEM((CH,), jnp.float32),
            pltpu.VMEM((CH,), jnp.float32),
            pltpu.SemaphoreType.DMA((4,)),
            pltpu.SemaphoreType.DMA((2,)),
            pltpu.SemaphoreType.DMA((2,)),
        ],
    )
    return fn(table, edges, zrows, z1, ones)


def _pad_edges(ei, n_edges, n_pad_total, n_src, n_dst_real, n_garbage, ch):
    src, dst = ei[0], ei[1]
    npad = n_pad_total - n_edges
    pad_i = jnp.arange(npad, dtype=jnp.int32)
    src = jnp.concatenate([src, pad_i % n_src])
    dst = jnp.concatenate([dst, n_dst_real + pad_i % n_garbage])
    return jnp.stack([src.reshape(-1, ch), dst.reshape(-1, ch)], axis=1)


def kernel(x_supplier, x_product, x_warehouse, ei_supplies, ei_stored,
           W_sup, b_sup, W_prod, b_prod, W_wh, b_wh,
           l0_sup_Wl, l0_sup_bl, l0_sup_Wr, l0_st_Wl, l0_st_bl, l0_st_Wr,
           l1_sup_Wl, l1_sup_bl, l1_sup_Wr, l1_st_Wl, l1_st_bl, l1_st_Wr):
    h_sup = _proj(x_supplier, W_sup, b_sup)
    h_prod0 = _proj(x_product, W_prod, b_prod)
    h_wh0 = _proj(x_warehouse, W_wh, b_wh)

    edgesA = _pad_edges(ei_supplies, ei_supplies.shape[1], E1P,
                        N_SUP, N_PROD, RP - N_PROD, CH)
    edgesB = _pad_edges(ei_stored, ei_stored.shape[1], E2P,
                        N_PROD, N_WH, RW - N_WH, CH)

    zrowsA = jnp.zeros((CH, 32), jnp.float32)
    zrowsB = jnp.zeros((CH, H), jnp.float32)
    z1 = jnp.zeros((CH,), jnp.float32)
    ones = jnp.ones((CH,), jnp.float32)

    tables = tuple(h_sup[:, f * 32:(f + 1) * 32] for f in range(4))
    psumA, pcntA = _agg_sup(tables, edgesA, zrowsA, z1, ones)
    cA = pcntA.reshape(RP, 1)

    sumB0, cntB0 = _agg_st(h_prod0, edgesB, zrowsB, z1, ones)

    h_prod1 = _combine1(psumA, cA, h_prod0,
                        l0_sup_Wl, l0_sup_Wr, l0_sup_bl)
    h_wh1 = _combine(sumB0[0], sumB0[1], cntB0[0].reshape(RW, 1),
                     cntB0[1].reshape(RW, 1), h_wh0,
                     l0_st_Wl, l0_st_Wr, l0_st_bl)

    sumB1, cntB1 = _agg_st(h_prod1, edgesB, zrowsB, z1, ones)

    h_prod2 = _combine1(psumA, cA, h_prod1,
                        l1_sup_Wl, l1_sup_Wr, l1_sup_bl)
    h_wh2 = _combine(sumB1[0], sumB1[1], cntB1[0].reshape(RW, 1),
                     cntB1[1].reshape(RW, 1), h_wh1,
                     l1_st_Wl, l1_st_Wr, l1_st_bl)

    return (h_sup, h_prod2, h_wh2)
